# baseline (speedup 1.0000x reference)
import functools

import jax
import jax.numpy as jnp
from jax import lax
from jax.experimental import pallas as pl
from jax.experimental.pallas import tpu as pltpu
from jax.experimental.pallas import tpu_sc as plsc

_G = 8
_TILE = 128


@functools.cache
def _sc_geometry():
    info = plsc.get_sparse_core_info()
    return info.num_cores, info.num_subcores


@functools.partial(jax.jit, static_argnums=(2, 3))
def _gather_cols(table_t, idx, b_per_w, nc):
    C, V = table_t.shape
    B = idx.shape[0]
    mesh = plsc.VectorSubcoreMesh(core_axis_name="c", subcore_axis_name="s")
    ngroups = b_per_w // _G

    @functools.partial(
        pl.kernel,
        mesh=mesh,
        out_type=jax.ShapeDtypeStruct((C, B), jnp.float32),
        scratch_types=[
            pltpu.VMEM((b_per_w + 16,), jnp.int32),
            pltpu.VMEM((2 * _G, C, _TILE), jnp.float32),
            pltpu.VMEM((C, b_per_w), jnp.float32),
            pltpu.SemaphoreType.DMA,
            pltpu.SemaphoreType.DMA,
            pltpu.SemaphoreType.DMA,
        ],
        compiler_params=pltpu.CompilerParams(needs_layout_passes=False),
    )
    def k(tab, idx_hbm, out, idx_v, tiles, obuf, sem0, sem1, osem):
        wid = lax.axis_index("s") * nc + lax.axis_index("c")
        base = wid * b_per_w
        pltpu.sync_copy(idx_hbm.at[pl.ds(base, b_per_w)], idx_v.at[pl.ds(0, b_per_w)])
        iota = lax.iota(jnp.int32, 16)
        row_halves = [iota + 16 * h for h in range(C // 16)]
        sems = (sem0, sem1)

        def fetch(g, phase, sem):
            vvec = idx_v[pl.ds(g * _G, 16)]
            for b in range(_G):
                off = pl.multiple_of(vvec[b] & -128, _TILE)
                pltpu.async_copy(
                    tab.at[:, pl.ds(off, _TILE)], tiles.at[phase * _G + b], sem
                )

        def drain(phase, sem):
            for b in range(_G):
                pltpu.make_async_copy(
                    tab.at[:, pl.ds(0, _TILE)], tiles.at[phase * _G + b], sem
                ).wait()

        def extract(g, phase):
            vvec = idx_v[pl.ds(g * _G, 16)]
            ovec = vvec & 127
            gb = g * _G
            for b in range(_G):
                col = jnp.broadcast_to(ovec[b], (16,))
                dst_col = jnp.broadcast_to((gb + b).astype(jnp.int32), (16,))
                for rows in row_halves:
                    vals = plsc.load_gather(tiles.at[phase * _G + b], [rows, col])
                    plsc.store_scatter(obuf, [rows, dst_col], vals)

        fetch(jnp.int32(0), 0, sem0)

        @pl.loop(0, ngroups // 2)
        def _(p):
            for phase in range(2):
                g = p * 2 + phase
                nsem = sems[1 - phase]

                @pl.when(g + 1 < ngroups)
                def _():
                    fetch(g + 1, 1 - phase, nsem)

                drain(phase, sems[phase])
                extract(g, phase)

        for t in range(b_per_w // _TILE):
            ocol = pl.multiple_of(base + t * _TILE, _TILE)
            pltpu.async_copy(
                obuf.at[:, pl.ds(t * _TILE, _TILE)],
                out.at[:, pl.ds(ocol, _TILE)],
                osem,
            ).wait()

    return k(table_t, idx)


def kernel(boxes, box_indices):
    nm, nb, two, dims = boxes.shape
    C = two * dims
    B = box_indices.shape[0]
    nc, ns = _sc_geometry()
    nw = nc * ns
    b_per_w = B // nw
    table_t = boxes.reshape(nb, C).T
    idx = box_indices.astype(jnp.int32)
    out_t = _gather_cols(table_t, idx, b_per_w, nc)
    order = jnp.argsort(idx)
    sv = idx[order]
    fudge = (sv[0] * 0 + order[0] * 0).astype(jnp.float32)
    out_t = out_t + fudge
    return out_t.reshape(nm, two, dims, B).transpose(0, 3, 1, 2)
